# baseline (speedup 1.0000x reference)
import functools

import jax
import jax.numpy as jnp
from jax import lax
from jax.experimental import pallas as pl
from jax.experimental.pallas import tpu as pltpu
from jax.experimental.pallas import tpu_sc as plsc

_SEQ = 28
_BATCH = 4096
_D = 1024
_NC = 2
_NS = 16
_NW = _NC * _NS
_ROWS_PER_WJ = _BATCH // _NW
_CHUNK = 16
_CPJ = _ROWS_PER_WJ // _CHUNK
_T = _SEQ * _CPJ
_NBUF = 2


def _row_base(t, wid):
    j = t // _CPJ
    c = t % _CPJ
    return j, j * _BATCH + wid * _ROWS_PER_WJ + c * _CHUNK


def _sc_body(x_hbm, pe_hbm, inds_hbm, out_hbm,
             idx_v, fpe_v, inb_v, outb_v, sg, sin0, sin1, sout0, sout1):
    sin = (sin0, sin1)
    sout = (sout0, sout1)
    wid = lax.axis_index("s") * _NC + lax.axis_index("c")

    pltpu.sync_copy(inds_hbm, idx_v)
    pltpu.async_copy(pe_hbm.at[idx_v], fpe_v, sg).wait()

    def chunk(t, _):
        j, base = _row_base(t, wid)
        pltpu.sync_copy(x_hbm.at[pl.ds(base, _CHUNK)], inb_v.at[0])

        def kloop(k, _):
            sl = pl.ds(k * 16, 16)
            row = fpe_v[j, sl]
            for r in range(_CHUNK):
                outb_v[0, r, sl] = inb_v[0, r, sl] + row
            return 0

        lax.fori_loop(0, _D // 16, kloop, 0)
        pltpu.sync_copy(outb_v.at[0], out_hbm.at[pl.ds(base, _CHUNK)])
        return 0

    lax.fori_loop(0, _T, chunk, 0)


@jax.jit
def _sc_add(x2, pe, inds32):
    mesh = plsc.VectorSubcoreMesh(
        core_axis_name="c", subcore_axis_name="s",
        num_cores=_NC, num_subcores=_NS,
    )
    return pl.kernel(
        _sc_body,
        out_type=jax.ShapeDtypeStruct((_SEQ * _BATCH, _D), jnp.float32),
        mesh=mesh,
        scratch_types=[
            pltpu.VMEM((32,), jnp.int32),
            pltpu.VMEM((32, _D), jnp.float32),
            pltpu.VMEM((_NBUF, _CHUNK, _D), jnp.float32),
            pltpu.VMEM((_NBUF, _CHUNK, _D), jnp.float32),
            pltpu.SemaphoreType.DMA,
            pltpu.SemaphoreType.DMA,
            pltpu.SemaphoreType.DMA,
            pltpu.SemaphoreType.DMA,
            pltpu.SemaphoreType.DMA,
        ],
    )(x2, pe, inds32)


def kernel(x, pe, inds):
    batch, seq, d_model = x.shape
    inds32 = jnp.concatenate(
        [inds.astype(jnp.int32), jnp.zeros((32 - seq,), jnp.int32)])
    xt = jnp.transpose(x, (1, 0, 2)).reshape(seq * batch, d_model)
    out2 = _sc_add(xt, pe, inds32)
    return jnp.transpose(out2.reshape(seq, batch, d_model), (1, 0, 2))


# speedup vs baseline: 2.0469x; 2.0469x over previous
import functools

import jax
import jax.numpy as jnp
from jax import lax
from jax.experimental import pallas as pl
from jax.experimental.pallas import tpu as pltpu
from jax.experimental.pallas import tpu_sc as plsc

_SEQ = 28
_BATCH = 4096
_D = 1024
_NC = 2
_NS = 16
_NW = _NC * _NS
_ROWS_PER_WJ = _BATCH // _NW
_CHUNK = 16
_CPJ = _ROWS_PER_WJ // _CHUNK
_T = _SEQ * _CPJ
_NBUF = 2


def _row_base(t, wid):
    j = t // _CPJ
    c = t % _CPJ
    return j, j * _BATCH + wid * _ROWS_PER_WJ + c * _CHUNK


def _sc_body(x_hbm, pe_hbm, inds_hbm, out_hbm,
             idx_v, fpe_v, inb_v, outb_v, sg, sin0, sin1, sout0, sout1):
    sin = (sin0, sin1)
    sout = (sout0, sout1)
    wid = lax.axis_index("s") * _NC + lax.axis_index("c")

    pltpu.sync_copy(inds_hbm, idx_v)
    pltpu.async_copy(pe_hbm.at[idx_v], fpe_v, sg).wait()

    for b in range(_NBUF):
        _, base = _row_base(b, wid)
        pltpu.async_copy(x_hbm.at[pl.ds(base, _CHUNK)], inb_v.at[b], sin[b])

    def group(g, _):
        for b in range(_NBUF):
            t = g * _NBUF + b
            j, base = _row_base(t, wid)

            @pl.when(g > 0)
            def _():
                pltpu.make_async_copy(
                    outb_v.at[b], out_hbm.at[pl.ds(base, _CHUNK)], sout[b]
                ).wait()

            pltpu.make_async_copy(
                x_hbm.at[pl.ds(base, _CHUNK)], inb_v.at[b], sin[b]
            ).wait()

            def kloop(k, _):
                sl = pl.ds(k * 16, 16)
                row = fpe_v[j, sl]
                for r in range(_CHUNK):
                    outb_v[b, r, sl] = inb_v[b, r, sl] + row
                return 0

            lax.fori_loop(0, _D // 16, kloop, 0)

            pltpu.async_copy(
                outb_v.at[b], out_hbm.at[pl.ds(base, _CHUNK)], sout[b]
            )

            @pl.when(t + _NBUF < _T)
            def _():
                _, base2 = _row_base(t + _NBUF, wid)
                pltpu.async_copy(
                    x_hbm.at[pl.ds(base2, _CHUNK)], inb_v.at[b], sin[b]
                )
        return 0

    lax.fori_loop(0, _T // _NBUF, group, 0)

    for b in range(_NBUF):
        pltpu.make_async_copy(
            outb_v.at[b], out_hbm.at[pl.ds(0, _CHUNK)], sout[b]
        ).wait()


@jax.jit
def _sc_add(x2, pe, inds32):
    mesh = plsc.VectorSubcoreMesh(
        core_axis_name="c", subcore_axis_name="s",
        num_cores=_NC, num_subcores=_NS,
    )
    return pl.kernel(
        _sc_body,
        out_type=jax.ShapeDtypeStruct((_SEQ * _BATCH, _D), jnp.float32),
        mesh=mesh,
        scratch_types=[
            pltpu.VMEM((32,), jnp.int32),
            pltpu.VMEM((32, _D), jnp.float32),
            pltpu.VMEM((_NBUF, _CHUNK, _D), jnp.float32),
            pltpu.VMEM((_NBUF, _CHUNK, _D), jnp.float32),
            pltpu.SemaphoreType.DMA,
            pltpu.SemaphoreType.DMA,
            pltpu.SemaphoreType.DMA,
            pltpu.SemaphoreType.DMA,
            pltpu.SemaphoreType.DMA,
        ],
    )(x2, pe, inds32)


def kernel(x, pe, inds):
    batch, seq, d_model = x.shape
    inds32 = jnp.concatenate(
        [inds.astype(jnp.int32), jnp.zeros((32 - seq,), jnp.int32)])
    xt = jnp.transpose(x, (1, 0, 2)).reshape(seq * batch, d_model)
    out2 = _sc_add(xt, pe, inds32)
    return jnp.transpose(out2.reshape(seq, batch, d_model), (1, 0, 2))


# speedup vs baseline: 2.4507x; 1.1973x over previous
import jax
import jax.numpy as jnp
from jax import lax
from jax.experimental import pallas as pl
from jax.experimental.pallas import tpu as pltpu
from jax.experimental.pallas import tpu_sc as plsc

_SEQ = 28
_BATCH = 4096
_D = 1024
_S_SC = 11
_NC = 2
_NS = 16
_NW = _NC * _NS
_ROWS_PER_WJ = _BATCH // _NW
_CHUNK = 16
_CPJ = _ROWS_PER_WJ // _CHUNK
_T = _S_SC * _CPJ
_NBUF = 2
_TC_BATCH_BLK = 2048


def _row_base(t, wid):
    j = t // _CPJ
    c = t % _CPJ
    return j, j * _BATCH + wid * _ROWS_PER_WJ + c * _CHUNK


def _sc_body(x_hbm, pe_hbm, inds_hbm, out_hbm,
             idx_v, fpe_v, inb_v, outb_v, sg, sin0, sin1, sout0, sout1):
    sin = (sin0, sin1)
    sout = (sout0, sout1)
    wid = lax.axis_index("s") * _NC + lax.axis_index("c")

    pltpu.sync_copy(inds_hbm, idx_v)
    pltpu.async_copy(pe_hbm.at[idx_v], fpe_v, sg).wait()

    for b in range(_NBUF):
        _, base = _row_base(b, wid)
        pltpu.async_copy(x_hbm.at[pl.ds(base, _CHUNK)], inb_v.at[b], sin[b])

    def group(g, _):
        for b in range(_NBUF):
            t = g * _NBUF + b
            j, base = _row_base(t, wid)

            @pl.when(g > 0)
            def _():
                pltpu.make_async_copy(
                    outb_v.at[b], out_hbm.at[pl.ds(base, _CHUNK)], sout[b]
                ).wait()

            pltpu.make_async_copy(
                x_hbm.at[pl.ds(base, _CHUNK)], inb_v.at[b], sin[b]
            ).wait()

            def kloop(k, _):
                sl = pl.ds(k * 16, 16)
                row = fpe_v[j, sl]
                for r in range(_CHUNK):
                    outb_v[b, r, sl] = inb_v[b, r, sl] + row
                return 0

            lax.fori_loop(0, _D // 16, kloop, 0)

            pltpu.async_copy(
                outb_v.at[b], out_hbm.at[pl.ds(base, _CHUNK)], sout[b]
            )

            @pl.when(t + _NBUF < _T)
            def _():
                _, base2 = _row_base(t + _NBUF, wid)
                pltpu.async_copy(
                    x_hbm.at[pl.ds(base2, _CHUNK)], inb_v.at[b], sin[b]
                )
        return 0

    lax.fori_loop(0, _T // _NBUF, group, 0)

    for b in range(_NBUF):
        pltpu.make_async_copy(
            outb_v.at[b], out_hbm.at[pl.ds(0, _CHUNK)], sout[b]
        ).wait()


def _sc_add(x2, pe, inds32):
    mesh = plsc.VectorSubcoreMesh(
        core_axis_name="c", subcore_axis_name="s",
        num_cores=_NC, num_subcores=_NS,
    )
    return pl.kernel(
        _sc_body,
        out_type=jax.ShapeDtypeStruct((_SEQ * _BATCH, _D), jnp.float32),
        mesh=mesh,
        scratch_types=[
            pltpu.VMEM((32,), jnp.int32),
            pltpu.VMEM((32, _D), jnp.float32),
            pltpu.VMEM((_NBUF, _CHUNK, _D), jnp.float32),
            pltpu.VMEM((_NBUF, _CHUNK, _D), jnp.float32),
            pltpu.SemaphoreType.DMA,
            pltpu.SemaphoreType.DMA,
            pltpu.SemaphoreType.DMA,
            pltpu.SemaphoreType.DMA,
            pltpu.SemaphoreType.DMA,
        ],
    )(x2, pe, inds32)


def _tc_body(inds_ref, sc_ref, x_ref, pe_ref, o_ref):
    del inds_ref, sc_ref
    o_ref[...] = x_ref[...] + pe_ref[...]


def _tc_fill(sc_out3, xt, pe3, inds32):
    n_seq = _SEQ - _S_SC
    grid = (n_seq, _BATCH // _TC_BATCH_BLK)
    return pl.pallas_call(
        _tc_body,
        grid_spec=pltpu.PrefetchScalarGridSpec(
            num_scalar_prefetch=1,
            grid=grid,
            in_specs=[
                pl.BlockSpec(memory_space=pl.ANY),
                pl.BlockSpec((1, _TC_BATCH_BLK, _D),
                             lambda j, i, inds_ref: (j + _S_SC, i, 0)),
                pl.BlockSpec((1, 1, _D),
                             lambda j, i, inds_ref: (inds_ref[j + _S_SC], 0, 0)),
            ],
            out_specs=pl.BlockSpec((1, _TC_BATCH_BLK, _D),
                                   lambda j, i, inds_ref: (j + _S_SC, i, 0)),
        ),
        out_shape=jax.ShapeDtypeStruct((_SEQ, _BATCH, _D), jnp.float32),
        input_output_aliases={1: 0},
        compiler_params=pltpu.CompilerParams(
            dimension_semantics=("arbitrary", "arbitrary"),
        ),
    )(inds32, sc_out3, xt, pe3)


@jax.jit
def _hybrid(x, pe, inds32):
    xt = jnp.transpose(x, (1, 0, 2))
    x2 = xt.reshape(_SEQ * _BATCH, _D)
    pe3 = pe.reshape(pe.shape[0], 1, _D)

    sc_out3 = _sc_add(x2, pe, inds32).reshape(_SEQ, _BATCH, _D)
    out_t = _tc_fill(sc_out3, xt, pe3, inds32)
    return jnp.transpose(out_t, (1, 0, 2))


def kernel(x, pe, inds):
    batch, seq, d_model = x.shape
    inds32 = jnp.concatenate(
        [inds.astype(jnp.int32), jnp.zeros((32 - seq,), jnp.int32)])
    return _hybrid(x, pe, inds32)


# speedup vs baseline: 2.9000x; 1.1833x over previous
import jax
import jax.numpy as jnp
from jax import lax
from jax.experimental import pallas as pl
from jax.experimental.pallas import tpu as pltpu
from jax.experimental.pallas import tpu_sc as plsc

_SEQ = 28
_BATCH = 4096
_D = 1024
_NC = 2
_NS = 16
_TC_BATCH_BLK = 2048


def _gather_body(pe_hbm, inds_hbm, out_hbm, idx_v, fpe_v, sg, so):
    wid = lax.axis_index("s") * _NC + lax.axis_index("c")

    @pl.when(wid == 0)
    def _():
        pltpu.sync_copy(inds_hbm, idx_v)
        pltpu.async_copy(pe_hbm.at[idx_v], fpe_v, sg).wait()
        pltpu.async_copy(fpe_v, out_hbm, so).wait()


def _sc_gather(pe, inds32):
    mesh = plsc.VectorSubcoreMesh(
        core_axis_name="c", subcore_axis_name="s",
        num_cores=_NC, num_subcores=_NS,
    )
    return pl.kernel(
        _gather_body,
        out_type=jax.ShapeDtypeStruct((32, _D), jnp.float32),
        mesh=mesh,
        scratch_types=[
            pltpu.VMEM((32,), jnp.int32),
            pltpu.VMEM((32, _D), jnp.float32),
            pltpu.SemaphoreType.DMA,
            pltpu.SemaphoreType.DMA,
        ],
    )(pe, inds32)


def _tc_body(x_ref, fpe_ref, o_ref):
    o_ref[...] = x_ref[...] + fpe_ref[...]


def _tc_add(xt, fpe):
    grid = (_SEQ, _BATCH // _TC_BATCH_BLK)
    return pl.pallas_call(
        _tc_body,
        grid=grid,
        in_specs=[
            pl.BlockSpec((1, _TC_BATCH_BLK, _D), lambda j, i: (j, i, 0)),
            pl.BlockSpec((1, 1, _D), lambda j, i: (j, 0, 0)),
        ],
        out_specs=pl.BlockSpec((1, _TC_BATCH_BLK, _D), lambda j, i: (j, i, 0)),
        out_shape=jax.ShapeDtypeStruct((_SEQ, _BATCH, _D), jnp.float32),
        compiler_params=pltpu.CompilerParams(
            dimension_semantics=("arbitrary", "arbitrary"),
        ),
    )(xt, fpe)


@jax.jit
def _run(x, pe, inds32):
    xt = jnp.transpose(x, (1, 0, 2))
    fpe = _sc_gather(pe, inds32)
    out_t = _tc_add(xt, fpe.reshape(32, 1, _D))
    return jnp.transpose(out_t, (1, 0, 2))


def kernel(x, pe, inds):
    batch, seq, d_model = x.shape
    inds32 = jnp.concatenate(
        [inds.astype(jnp.int32), jnp.zeros((32 - seq,), jnp.int32)])
    return _run(x, pe, inds32)


# speedup vs baseline: 2.9155x; 1.0054x over previous
import jax
import jax.numpy as jnp
from jax import lax
from jax.experimental import pallas as pl
from jax.experimental.pallas import tpu as pltpu
from jax.experimental.pallas import tpu_sc as plsc

_SEQ = 28
_BATCH = 4096
_D = 1024
_NC = 2
_NS = 16
_TC_BATCH_BLK = 2048


def _gather_body(pe_hbm, inds_hbm, out_hbm, idx_v, fpe_v, sg, so):
    wid = lax.axis_index("s") * _NC + lax.axis_index("c")

    @pl.when(wid == 0)
    def _():
        pltpu.sync_copy(inds_hbm, idx_v)
        pltpu.async_copy(pe_hbm.at[idx_v], fpe_v, sg).wait()
        pltpu.async_copy(fpe_v, out_hbm, so).wait()


def _sc_gather(pe, inds32):
    mesh = plsc.VectorSubcoreMesh(
        core_axis_name="c", subcore_axis_name="s",
        num_cores=1, num_subcores=1,
    )
    return pl.kernel(
        _gather_body,
        out_type=jax.ShapeDtypeStruct((32, _D), jnp.float32),
        mesh=mesh,
        scratch_types=[
            pltpu.VMEM((32,), jnp.int32),
            pltpu.VMEM((32, _D), jnp.float32),
            pltpu.SemaphoreType.DMA,
            pltpu.SemaphoreType.DMA,
        ],
    )(pe, inds32)


def _tc_body(x_ref, fpe_ref, o_ref):
    o_ref[...] = x_ref[...] + fpe_ref[...]


def _tc_add(xt, fpe):
    grid = (_SEQ, _BATCH // _TC_BATCH_BLK)
    return pl.pallas_call(
        _tc_body,
        grid=grid,
        in_specs=[
            pl.BlockSpec((1, _TC_BATCH_BLK, _D), lambda j, i: (j, i, 0)),
            pl.BlockSpec((1, 1, _D), lambda j, i: (j, 0, 0)),
        ],
        out_specs=pl.BlockSpec((1, _TC_BATCH_BLK, _D), lambda j, i: (j, i, 0)),
        out_shape=jax.ShapeDtypeStruct((_SEQ, _BATCH, _D), jnp.float32),
        compiler_params=pltpu.CompilerParams(
            dimension_semantics=("arbitrary", "arbitrary"),
        ),
    )(xt, fpe)


@jax.jit
def _run(x, pe, inds32):
    xt = jnp.transpose(x, (1, 0, 2))
    fpe = _sc_gather(pe, inds32)
    out_t = _tc_add(xt, fpe.reshape(32, 1, _D))
    return jnp.transpose(out_t, (1, 0, 2))


def kernel(x, pe, inds):
    batch, seq, d_model = x.shape
    inds32 = jnp.concatenate(
        [inds.astype(jnp.int32), jnp.zeros((32 - seq,), jnp.int32)])
    return _run(x, pe, inds32)
